# baseline (speedup 1.0000x reference)
import jax
import jax.numpy as jnp
from jax.experimental import pallas as pl


def _copy_body(x_ref, ea_ref, xo_ref, eao_ref):
    xo_ref[...] = x_ref[...]
    eao_ref[...] = ea_ref[...]


def kernel(x, x_lstm, encoded_z_gnss, edge_index, edge_attr,
           node_indexes_related_to_agent, edge_indexes_related_to_agent):
    N, DF = x.shape
    E, DE = edge_attr.shape
    eat = edge_attr.T
    G = 1
    xn, ean = pl.pallas_call(
        _copy_body,
        grid=(G,),
        in_specs=[
            pl.BlockSpec((N // G, DF), lambda i: (i, 0)),
            pl.BlockSpec((DE, E // G), lambda i: (0, i)),
        ],
        out_specs=[
            pl.BlockSpec((N // G, DF), lambda i: (i, 0)),
            pl.BlockSpec((DE, E // G), lambda i: (0, i)),
        ],
        out_shape=[
            jax.ShapeDtypeStruct((N, DF), x.dtype),
            jax.ShapeDtypeStruct((DE, E), edge_attr.dtype),
        ],
    )(x, eat)
    return (xn, ean.T)


# speedup vs baseline: 1.1385x; 1.1385x over previous
import jax
import jax.numpy as jnp
from jax.experimental import pallas as pl


def _copy_body(x_ref, ea_ref, xo_ref, eao_ref):
    xo_ref[...] = x_ref[...]
    eao_ref[...] = ea_ref[...]


def kernel(x, x_lstm, encoded_z_gnss, edge_index, edge_attr,
           node_indexes_related_to_agent, edge_indexes_related_to_agent):
    N, DF = x.shape
    E, DE = edge_attr.shape
    eat = edge_attr.T
    G = 2
    xn, ean = pl.pallas_call(
        _copy_body,
        grid=(G,),
        in_specs=[
            pl.BlockSpec((N // G, DF), lambda i: (i, 0)),
            pl.BlockSpec((DE, E // G), lambda i: (0, i)),
        ],
        out_specs=[
            pl.BlockSpec((N // G, DF), lambda i: (i, 0)),
            pl.BlockSpec((DE, E // G), lambda i: (0, i)),
        ],
        out_shape=[
            jax.ShapeDtypeStruct((N, DF), x.dtype),
            jax.ShapeDtypeStruct((DE, E), edge_attr.dtype),
        ],
    )(x, eat)
    return (xn, ean.T)


# speedup vs baseline: 1.1440x; 1.0049x over previous
import jax
from jax.experimental import pallas as pl


def _copy_body(x_ref, ea_ref, xo_ref, eao_ref):
    xo_ref[...] = x_ref[...]
    eao_ref[...] = ea_ref[...]


def kernel(x, x_lstm, encoded_z_gnss, edge_index, edge_attr,
           node_indexes_related_to_agent, edge_indexes_related_to_agent):
    N, DF = x.shape
    E, DE = edge_attr.shape
    eat = edge_attr.T
    G = 2
    xn, ean = pl.pallas_call(
        _copy_body,
        grid=(G,),
        in_specs=[
            pl.BlockSpec((N // G, DF), lambda i: (i, 0)),
            pl.BlockSpec((DE, E // G), lambda i: (0, i)),
        ],
        out_specs=[
            pl.BlockSpec((N // G, DF), lambda i: (i, 0)),
            pl.BlockSpec((DE, E // G), lambda i: (0, i)),
        ],
        out_shape=[
            jax.ShapeDtypeStruct((N, DF), x.dtype),
            jax.ShapeDtypeStruct((DE, E), edge_attr.dtype),
        ],
    )(x, eat)
    return (xn, ean.T)
